# Initial kernel scaffold; baseline (speedup 1.0000x reference)
import jax
import jax.numpy as jnp
from jax.experimental import pallas as pl


def kernel(indices, features):
    raise NotImplementedError("write your pallas kernel here")


# speedup vs baseline: 4.2543x; 4.2543x over previous
import functools

import jax
import jax.numpy as jnp
from jax import lax
from jax.experimental import pallas as pl
from jax.experimental.pallas import tpu as pltpu
from jax.experimental.pallas import tpu_sc as plsc

L = 16


def _build_sc_gather(n_idx: int, n_rows: int, d: int):
    info = plsc.get_sparse_core_info()
    nc, ns = info.num_cores, info.num_subcores
    nw = nc * ns

    per_w = n_idx // nw
    assert per_w * nw == n_idx
    chunk = 2048
    assert per_w % chunk == 0
    n_chunks = per_w // chunk
    groups = chunk // L

    mesh = plsc.VectorSubcoreMesh(core_axis_name="c", subcore_axis_name="s")

    @functools.partial(
        pl.kernel,
        mesh=mesh,
        compiler_params=pltpu.CompilerParams(needs_layout_passes=False),
        out_type=jax.ShapeDtypeStruct((n_idx * d,), jnp.float32),
        scratch_types=[
            pltpu.VMEM((n_rows * d,), jnp.float32),
            pltpu.VMEM((chunk,), jnp.int32),
            pltpu.VMEM((chunk * d,), jnp.float32),
        ],
    )
    def sc_gather(idx_hbm, tbl_hbm, out_hbm, tbl_v, idx_v, out_v):
        wid = lax.axis_index("s") * nc + lax.axis_index("c")
        base = wid * per_w
        pltpu.sync_copy(tbl_hbm, tbl_v)
        lane_d = lax.iota(jnp.int32, L) * d

        def chunk_body(ci, _):
            off = base + ci * chunk
            pltpu.sync_copy(idx_hbm.at[pl.ds(off, chunk)], idx_v)

            def group_body(g, _):
                idx16 = idx_v[pl.ds(g * L, L)]
                addr = idx16 * d
                pos = g * (L * d) + lane_d
                for f in range(d):
                    vals = plsc.load_gather(tbl_v, [addr + f])
                    plsc.store_scatter(out_v, [pos + f], vals)
                return 0

            lax.fori_loop(0, groups, group_body, 0, unroll=False)
            pltpu.sync_copy(out_v, out_hbm.at[pl.ds(off * d, chunk * d)])
            return 0

        lax.fori_loop(0, n_chunks, chunk_body, 0, unroll=False)

    return sc_gather


def kernel(indices, features):
    b, hl = indices.shape
    v, d = features.shape
    idx_flat = indices.reshape(-1).astype(jnp.int32)
    tbl_flat = features.reshape(-1)
    fn = _build_sc_gather(idx_flat.shape[0], v, d)
    out = fn(idx_flat, tbl_flat)
    return out.reshape(b, hl, d)


# speedup vs baseline: 4.9054x; 1.1530x over previous
import functools

import jax
import jax.numpy as jnp
from jax import lax
from jax.experimental import pallas as pl
from jax.experimental.pallas import tpu as pltpu
from jax.experimental.pallas import tpu_sc as plsc

L = 16


def _build_sc_gather(n_idx: int, n_rows: int, d: int):
    info = plsc.get_sparse_core_info()
    nc, ns = info.num_cores, info.num_subcores
    nw = nc * ns

    per_w = n_idx // nw
    assert per_w * nw == n_idx
    chunk = 3200
    assert per_w % (2 * chunk) == 0
    n_chunks = per_w // chunk
    groups = chunk // L

    mesh = plsc.VectorSubcoreMesh(core_axis_name="c", subcore_axis_name="s")

    @functools.partial(
        pl.kernel,
        mesh=mesh,
        compiler_params=pltpu.CompilerParams(needs_layout_passes=False),
        out_type=jax.ShapeDtypeStruct((n_idx * d,), jnp.float32),
        scratch_types=[
            pltpu.VMEM((n_rows * d,), jnp.float32),
            pltpu.VMEM((2, chunk), jnp.int32),
            pltpu.VMEM((2 * chunk * d,), jnp.float32),
            pltpu.SemaphoreType.DMA,
            pltpu.SemaphoreType.DMA,
            pltpu.SemaphoreType.DMA,
            pltpu.SemaphoreType.DMA,
        ],
    )
    def sc_gather(idx_hbm, tbl_hbm, out_hbm, tbl_v, idx_v, out_v,
                  sin0, sin1, sout0, sout1):
        sin = (sin0, sin1)
        sout = (sout0, sout1)
        wid = lax.axis_index("s") * nc + lax.axis_index("c")
        base = wid * per_w
        pltpu.sync_copy(tbl_hbm, tbl_v)
        lane_d = lax.iota(jnp.int32, L) * d

        for b in range(2):
            pltpu.async_copy(
                idx_hbm.at[pl.ds(base + b * chunk, chunk)], idx_v.at[b], sin[b]
            )

        def pair_body(cj, _):
            for b in range(2):
                ci = cj * 2 + b
                off = base + ci * chunk
                pltpu.make_async_copy(
                    idx_hbm.at[pl.ds(off, chunk)], idx_v.at[b], sin[b]
                ).wait()

                @pl.when(ci >= 2)
                def _wait_out():
                    pltpu.make_async_copy(
                        out_v.at[pl.ds(b * chunk * d, chunk * d)],
                        out_hbm.at[pl.ds((off - 2 * chunk) * d, chunk * d)],
                        sout[b],
                    ).wait()

                @plsc.parallel_loop(0, groups, 1, unroll=4)
                def group_body(g):
                    idx16 = idx_v[b, pl.ds(g * L, L)]
                    addr = idx16 * d
                    pos = b * (chunk * d) + g * (L * d) + lane_d
                    for f in range(d):
                        vals = plsc.load_gather(tbl_v, [addr + f])
                        plsc.store_scatter(out_v, [pos + f], vals)

                pltpu.async_copy(
                    out_v.at[pl.ds(b * chunk * d, chunk * d)],
                    out_hbm.at[pl.ds(off * d, chunk * d)],
                    sout[b],
                )

                @pl.when(ci + 2 < n_chunks)
                def _next_in():
                    pltpu.async_copy(
                        idx_hbm.at[pl.ds(off + 2 * chunk, chunk)],
                        idx_v.at[b],
                        sin[b],
                    )

            return 0

        lax.fori_loop(0, n_chunks // 2, pair_body, 0, unroll=False)

        for b in range(2):
            ci = n_chunks - 2 + b
            off = base + ci * chunk
            pltpu.make_async_copy(
                out_v.at[pl.ds(b * chunk * d, chunk * d)],
                out_hbm.at[pl.ds(off * d, chunk * d)],
                sout[b],
            ).wait()

    return sc_gather


def kernel(indices, features):
    b, hl = indices.shape
    v, d = features.shape
    idx_flat = indices.reshape(-1).astype(jnp.int32)
    tbl_flat = features.reshape(-1)
    fn = _build_sc_gather(idx_flat.shape[0], v, d)
    out = fn(idx_flat, tbl_flat)
    return out.reshape(b, hl, d)


# speedup vs baseline: 6.3260x; 1.2896x over previous
import functools

import jax
import jax.numpy as jnp
from jax import lax
from jax.experimental import pallas as pl
from jax.experimental.pallas import tpu as pltpu
from jax.experimental.pallas import tpu_sc as plsc

L = 16


def _build_sc_gather(n_idx: int, n_rows: int, d: int):
    info = plsc.get_sparse_core_info()
    nc, ns = info.num_cores, info.num_subcores
    nw = nc * ns

    per_w = n_idx // nw
    assert per_w * nw == n_idx
    chunk = 3200
    assert per_w % (2 * chunk) == 0
    n_chunks = per_w // chunk
    groups = chunk // L

    mesh = plsc.VectorSubcoreMesh(core_axis_name="c", subcore_axis_name="s")

    @functools.partial(
        pl.kernel,
        mesh=mesh,
        compiler_params=pltpu.CompilerParams(
            needs_layout_passes=False, use_tc_tiling_on_sc=False
        ),
        out_type=jax.ShapeDtypeStruct((n_idx, d), jnp.float32),
        scratch_types=[
            pltpu.VMEM((n_rows * d,), jnp.float32),
            pltpu.VMEM((2, chunk), jnp.int32),
            pltpu.VMEM((chunk, d), jnp.float32),
            pltpu.VMEM((chunk, d), jnp.float32),
            pltpu.SemaphoreType.DMA,
            pltpu.SemaphoreType.DMA,
            pltpu.SemaphoreType.DMA,
            pltpu.SemaphoreType.DMA,
        ],
    )
    def sc_gather(idx_hbm, tbl_hbm, out_hbm, tbl_v, idx_v, out_v0, out_v1,
                  sin0, sin1, sout0, sout1):
        sin = (sin0, sin1)
        sout = (sout0, sout1)
        out_v = (out_v0, out_v1)
        wid = lax.axis_index("s") * nc + lax.axis_index("c")
        base = wid * per_w
        pltpu.sync_copy(tbl_hbm, tbl_v)
        lane = lax.iota(jnp.int32, L)

        for b in range(2):
            pltpu.async_copy(
                idx_hbm.at[pl.ds(base + b * chunk, chunk)], idx_v.at[b], sin[b]
            )

        def pair_body(cj, _):
            for b in range(2):
                ci = cj * 2 + b
                off = base + ci * chunk
                pltpu.make_async_copy(
                    idx_hbm.at[pl.ds(off, chunk)], idx_v.at[b], sin[b]
                ).wait()

                @pl.when(ci >= 2)
                def _wait_out():
                    pltpu.make_async_copy(
                        out_v[b],
                        out_hbm.at[pl.ds(off - 2 * chunk, chunk)],
                        sout[b],
                    ).wait()

                @plsc.parallel_loop(0, groups, 1, unroll=4)
                def group_body(g):
                    idx16 = idx_v[b, pl.ds(g * L, L)]
                    addr = idx16 * d
                    n16 = g * L + lane
                    for f in range(d):
                        vals = plsc.load_gather(tbl_v, [addr + f])
                        plsc.store_scatter(
                            out_v[b], [n16, jnp.full((L,), f, jnp.int32)], vals
                        )

                pltpu.async_copy(
                    out_v[b], out_hbm.at[pl.ds(off, chunk)], sout[b]
                )

                @pl.when(ci + 2 < n_chunks)
                def _next_in():
                    pltpu.async_copy(
                        idx_hbm.at[pl.ds(off + 2 * chunk, chunk)],
                        idx_v.at[b],
                        sin[b],
                    )

            return 0

        lax.fori_loop(0, n_chunks // 2, pair_body, 0, unroll=False)

        for b in range(2):
            ci = n_chunks - 2 + b
            off = base + ci * chunk
            pltpu.make_async_copy(
                out_v[b], out_hbm.at[pl.ds(off, chunk)], sout[b]
            ).wait()

    return sc_gather


def kernel(indices, features):
    b, hl = indices.shape
    v, d = features.shape
    idx_flat = indices.reshape(-1).astype(jnp.int32)
    tbl_flat = features.reshape(-1)
    fn = _build_sc_gather(idx_flat.shape[0], v, d)
    out = fn(idx_flat, tbl_flat)
    return out.reshape(b, hl, d)


# speedup vs baseline: 135.4526x; 21.4119x over previous
import functools

import jax
import jax.numpy as jnp
from jax import lax
from jax.experimental import pallas as pl
from jax.experimental.pallas import tpu as pltpu
from jax.experimental.pallas import tpu_sc as plsc

L = 16


def _build_sc_gather(n_b: int, n_l: int, n_rows: int, d: int):
    info = plsc.get_sparse_core_info()
    nc, ns = info.num_cores, info.num_subcores
    nw = nc * ns

    bw = n_b // nw
    assert bw * nw == n_b
    lt_n = n_l // 8
    assert lt_n * 8 == n_l
    bc_w = 256
    bc_n = bw // bc_w
    assert bc_n * bc_w == bw and bc_n % 2 == 0
    pairs = 8 * (bc_w // L)

    mesh = plsc.VectorSubcoreMesh(core_axis_name="c", subcore_axis_name="s")

    @functools.partial(
        pl.kernel,
        mesh=mesh,
        compiler_params=pltpu.CompilerParams(needs_layout_passes=False),
        out_type=jax.ShapeDtypeStruct((d, n_l, n_b), jnp.float32),
        scratch_types=[
            pltpu.VMEM((n_rows * d,), jnp.float32),
            pltpu.VMEM((8, bc_w), jnp.int32),
            pltpu.VMEM((8, bc_w), jnp.int32),
            pltpu.VMEM((d, 8, bc_w), jnp.float32),
            pltpu.VMEM((d, 8, bc_w), jnp.float32),
            pltpu.SemaphoreType.DMA,
            pltpu.SemaphoreType.DMA,
            pltpu.SemaphoreType.DMA,
            pltpu.SemaphoreType.DMA,
        ],
    )
    def sc_gather(idx_hbm, tbl_hbm, out_hbm, tbl_v, idx_v0, idx_v1,
                  out_v0, out_v1, sin0, sin1, sout0, sout1):
        sin = (sin0, sin1)
        sout = (sout0, sout1)
        idx_v = (idx_v0, idx_v1)
        out_v = (out_v0, out_v1)
        wid = lax.axis_index("s") * nc + lax.axis_index("c")
        b0w = wid * bw
        pltpu.sync_copy(tbl_hbm, tbl_v)

        n_chunks = lt_n * bc_n

        def in_slice(ci):
            lt = ci // bc_n
            bc = ci % bc_n
            return idx_hbm.at[pl.ds(lt * 8, 8), pl.ds(b0w + bc * bc_w, bc_w)]

        def out_slice(ci):
            lt = ci // bc_n
            bc = ci % bc_n
            return out_hbm.at[:, pl.ds(lt * 8, 8), pl.ds(b0w + bc * bc_w, bc_w)]

        for b in range(2):
            pltpu.async_copy(in_slice(b), idx_v[b], sin[b])

        def pair_body(cj, _):
            for b in range(2):
                ci = cj * 2 + b
                pltpu.make_async_copy(in_slice(ci), idx_v[b], sin[b]).wait()

                @pl.when(ci >= 2)
                def _wait_out():
                    pltpu.make_async_copy(
                        out_v[b], out_slice(ci - 2), sout[b]
                    ).wait()

                @plsc.parallel_loop(0, pairs, 1, unroll=2)
                def group_body(p):
                    lr = p // (bc_w // L)
                    g = p % (bc_w // L)
                    idx16 = idx_v[b][lr, pl.ds(g * L, L)]
                    addr = idx16 * d
                    for f in range(d):
                        vals = plsc.load_gather(tbl_v, [addr + f])
                        out_v[b][f, lr, pl.ds(g * L, L)] = vals

                pltpu.async_copy(out_v[b], out_slice(ci), sout[b])

                @pl.when(ci + 2 < n_chunks)
                def _next_in():
                    pltpu.async_copy(in_slice(ci + 2), idx_v[b], sin[b])

            return 0

        lax.fori_loop(0, n_chunks // 2, pair_body, 0, unroll=False)

        for b in range(2):
            pltpu.make_async_copy(
                out_v[b], out_slice(n_chunks - 2 + b), sout[b]
            ).wait()

    return sc_gather


def kernel(indices, features):
    b, hl = indices.shape
    v, d = features.shape
    idx_t = jnp.transpose(indices.astype(jnp.int32))
    tbl_flat = features.reshape(-1)
    fn = _build_sc_gather(b, hl, v, d)
    out_t = fn(idx_t, tbl_flat)
    return jnp.transpose(out_t, (2, 1, 0))
